# speedup vs baseline: 2.9801x; 2.9801x over previous
import functools

import jax
import jax.numpy as jnp
from jax import lax
from jax.experimental import pallas as pl
from jax.experimental.pallas import tpu as pltpu
from jax.experimental.pallas import tpu_sc as plsc

EMB = 128
NC = 2
NS = 16
NW = NC * NS
CHUNK = 128


@functools.lru_cache(maxsize=None)
def _make_gather(B):
    b_per_w = B // NW
    nchunk = b_per_w // CHUNK
    mesh = plsc.VectorSubcoreMesh(core_axis_name="c", subcore_axis_name="s")

    @functools.partial(
        pl.kernel,
        mesh=mesh,
        out_type=jax.ShapeDtypeStruct((B, EMB), jnp.float32),
        scratch_types=[
            pltpu.VMEM((nchunk, CHUNK), jnp.int32),
            pltpu.VMEM((CHUNK, EMB), jnp.float32),
            pltpu.SemaphoreType.DMA,
        ],
    )
    def gather_kernel(idx_hbm, table_hbm, out_hbm, idx_v, rows_v, sem):
        wid = lax.axis_index("s") * NC + lax.axis_index("c")
        pltpu.sync_copy(idx_hbm.at[wid], idx_v)
        base = wid * b_per_w

        def chunk_body(i, carry):
            pltpu.async_copy(table_hbm.at[idx_v.at[i]], rows_v, sem).wait()
            pltpu.sync_copy(rows_v, out_hbm.at[pl.ds(base + i * CHUNK, CHUNK)])
            return carry

        lax.fori_loop(0, nchunk, chunk_body, 0)

    return gather_kernel


def kernel(input_tensor, table):
    n, s = input_tensor.shape
    B = n * s
    idx = input_tensor.reshape(NW, B // NW // CHUNK, CHUNK)
    out = _make_gather(B)(idx, table)
    return out.reshape(n, s, EMB)


# speedup vs baseline: 3.3447x; 1.1223x over previous
import functools

import jax
import jax.numpy as jnp
from jax import lax
from jax.experimental import pallas as pl
from jax.experimental.pallas import tpu as pltpu
from jax.experimental.pallas import tpu_sc as plsc

EMB = 128
NC = 2
NS = 16
NW = NC * NS
CHUNK = 128


@functools.lru_cache(maxsize=None)
def _make_gather(B):
    b_per_w = B // NW
    nchunk = b_per_w // CHUNK
    mesh = plsc.VectorSubcoreMesh(core_axis_name="c", subcore_axis_name="s")

    ngroup = nchunk // 2
    assert nchunk % 2 == 0 and ngroup >= 2

    @functools.partial(
        pl.kernel,
        mesh=mesh,
        out_type=jax.ShapeDtypeStruct((B, EMB), jnp.float32),
        scratch_types=[
            pltpu.VMEM((nchunk, CHUNK), jnp.int32),
            pltpu.VMEM((2, CHUNK, EMB), jnp.float32),
            pltpu.SemaphoreType.DMA,
            pltpu.SemaphoreType.DMA,
        ],
    )
    def gather_kernel(idx_hbm, table_hbm, out_hbm, idx_v, rows_v, sem0, sem1):
        wid = lax.axis_index("s") * NC + lax.axis_index("c")
        pltpu.sync_copy(idx_hbm.at[wid], idx_v)
        base = wid * b_per_w
        sems = (sem0, sem1)

        for b in range(2):
            pltpu.async_copy(table_hbm.at[idx_v.at[b]], rows_v.at[b], sems[b])

        def group_body(g, carry):
            for b in range(2):
                i = g * 2 + b
                pltpu.make_async_copy(
                    table_hbm.at[idx_v.at[b]], rows_v.at[b], sems[b]
                ).wait()
                pltpu.sync_copy(
                    rows_v.at[b], out_hbm.at[pl.ds(base + i * CHUNK, CHUNK)]
                )
                pltpu.async_copy(
                    table_hbm.at[idx_v.at[i + 2]], rows_v.at[b], sems[b]
                )
            return carry

        lax.fori_loop(0, ngroup - 1, group_body, 0)

        for b in range(2):
            i = (ngroup - 1) * 2 + b
            pltpu.make_async_copy(
                table_hbm.at[idx_v.at[b]], rows_v.at[b], sems[b]
            ).wait()
            pltpu.sync_copy(
                rows_v.at[b], out_hbm.at[pl.ds(base + i * CHUNK, CHUNK)]
            )

    return gather_kernel


def kernel(input_tensor, table):
    n, s = input_tensor.shape
    B = n * s
    idx = input_tensor.reshape(NW, B // NW // CHUNK, CHUNK)
    out = _make_gather(B)(idx, table)
    return out.reshape(n, s, EMB)


# speedup vs baseline: 3.3493x; 1.0014x over previous
import functools

import jax
import jax.numpy as jnp
from jax import lax
from jax.experimental import pallas as pl
from jax.experimental.pallas import tpu as pltpu
from jax.experimental.pallas import tpu_sc as plsc

EMB = 128
NC = 2
NS = 16
NW = NC * NS
CHUNK = 128
NBUF = 5
AHEAD = 2


@functools.lru_cache(maxsize=None)
def _make_gather(B):
    b_per_w = B // NW
    nchunk = b_per_w // CHUNK
    ngroup = nchunk // NBUF
    assert nchunk % NBUF == 0 and ngroup >= 3
    mesh = plsc.VectorSubcoreMesh(core_axis_name="c", subcore_axis_name="s")

    @functools.partial(
        pl.kernel,
        mesh=mesh,
        out_type=jax.ShapeDtypeStruct((B, EMB), jnp.float32),
        scratch_types=[
            pltpu.VMEM((nchunk, CHUNK), jnp.int32),
            pltpu.VMEM((NBUF, CHUNK, EMB), jnp.float32),
        ]
        + [pltpu.SemaphoreType.DMA] * (2 * NBUF),
    )
    def gather_kernel(idx_hbm, table_hbm, out_hbm, idx_v, rows_v, *sems):
        gsems, ssems = sems[:NBUF], sems[NBUF:]
        wid = lax.axis_index("s") * NC + lax.axis_index("c")
        pltpu.sync_copy(idx_hbm.at[wid], idx_v)
        base = wid * b_per_w

        def out_slice(i):
            return out_hbm.at[pl.ds(base + i * CHUNK, CHUNK)]

        def issue_gather(i, j):
            pltpu.async_copy(table_hbm.at[idx_v.at[i]], rows_v.at[j], gsems[j])

        def wait_gather(j):
            pltpu.make_async_copy(
                table_hbm.at[idx_v.at[0]], rows_v.at[j], gsems[j]
            ).wait()

        def issue_store(i, j):
            pltpu.async_copy(rows_v.at[j], out_slice(i), ssems[j])

        def wait_store(i, j):
            pltpu.make_async_copy(rows_v.at[j], out_slice(i), ssems[j]).wait()

        def step(i, b, do_wait, do_issue):
            s = (b + AHEAD) % NBUF
            if do_wait:
                wait_store(i + AHEAD - NBUF, s)
            if do_issue:
                issue_gather(i + AHEAD, s)
            wait_gather(b)
            issue_store(i, b)

        for j in range(AHEAD):
            issue_gather(j, j)

        for b in range(NBUF):
            step(b, b, do_wait=b + AHEAD >= NBUF, do_issue=True)

        def group_body(g, carry):
            for b in range(NBUF):
                step(g * NBUF + b, b, do_wait=True, do_issue=True)
            return carry

        lax.fori_loop(1, ngroup - 1, group_body, 0)

        for b in range(NBUF):
            i = (ngroup - 1) * NBUF + b
            step(i, b, do_wait=True, do_issue=i + AHEAD < nchunk)
        for i in range(nchunk - NBUF + AHEAD, nchunk):
            wait_store(i, i % NBUF)

    return gather_kernel


def kernel(input_tensor, table):
    n, s = input_tensor.shape
    B = n * s
    idx = input_tensor.reshape(NW, B // NW // CHUNK, CHUNK)
    out = _make_gather(B)(idx, table)
    return out.reshape(n, s, EMB)


# speedup vs baseline: 5.9703x; 1.7825x over previous
import functools

import jax
import jax.numpy as jnp
from jax import lax
from jax.experimental import pallas as pl
from jax.experimental.pallas import tpu as pltpu
from jax.experimental.pallas import tpu_sc as plsc

EMB = 128
NC = 2
NS = 16
NW = NC * NS
NBUF = 8
AHEAD = 4


@functools.lru_cache(maxsize=None)
def _make_gather(N, S):
    n_per_w = N // NW
    ngroup = n_per_w // NBUF
    assert n_per_w % NBUF == 0 and ngroup >= 3
    mesh = plsc.VectorSubcoreMesh(core_axis_name="c", subcore_axis_name="s")

    @functools.partial(
        pl.kernel,
        mesh=mesh,
        out_type=jax.ShapeDtypeStruct((N, S, EMB), jnp.float32),
        scratch_types=[
            pltpu.VMEM((n_per_w, S), jnp.int32),
            pltpu.VMEM((NBUF, S, EMB), jnp.float32),
        ]
        + [pltpu.SemaphoreType.DMA] * (2 * NBUF),
    )
    def gather_kernel(idx_hbm, table_hbm, out_hbm, idx_v, rows_v, *sems):
        gsems, ssems = sems[:NBUF], sems[NBUF:]
        wid = lax.axis_index("s") * NC + lax.axis_index("c")
        pltpu.sync_copy(idx_hbm.at[wid], idx_v)
        base = wid * n_per_w

        def issue_gather(i, j):
            pltpu.async_copy(table_hbm.at[idx_v.at[i]], rows_v.at[j], gsems[j])

        def wait_gather(j):
            pltpu.make_async_copy(
                table_hbm.at[idx_v.at[0]], rows_v.at[j], gsems[j]
            ).wait()

        def issue_store(i, j):
            pltpu.async_copy(rows_v.at[j], out_hbm.at[base + i], ssems[j])

        def wait_store(i, j):
            pltpu.make_async_copy(
                rows_v.at[j], out_hbm.at[base + i], ssems[j]
            ).wait()

        def step(i, b, do_wait, do_issue):
            s = (b + AHEAD) % NBUF
            if do_wait:
                wait_store(i + AHEAD - NBUF, s)
            if do_issue:
                issue_gather(i + AHEAD, s)
            wait_gather(b)
            issue_store(i, b)

        for j in range(AHEAD):
            issue_gather(j, j)

        for b in range(NBUF):
            step(b, b, do_wait=b + AHEAD >= NBUF, do_issue=True)

        def group_body(g, carry):
            for b in range(NBUF):
                step(g * NBUF + b, b, do_wait=True, do_issue=True)
            return carry

        lax.fori_loop(1, ngroup - 1, group_body, 0)

        for b in range(NBUF):
            i = (ngroup - 1) * NBUF + b
            step(i, b, do_wait=True, do_issue=i + AHEAD < n_per_w)
        for i in range(n_per_w - NBUF + AHEAD, n_per_w):
            wait_store(i, i % NBUF)

    return gather_kernel


def kernel(input_tensor, table):
    n, s = input_tensor.shape
    idx = input_tensor.reshape(NW, n // NW, s)
    return _make_gather(n, s)(idx, table)


# speedup vs baseline: 5.9711x; 1.0001x over previous
import functools

import jax
import jax.numpy as jnp
from jax import lax
from jax.experimental import pallas as pl
from jax.experimental.pallas import tpu as pltpu
from jax.experimental.pallas import tpu_sc as plsc

EMB = 128
NC = 2
NS = 16
NW = NC * NS
NBUF = 8
AHEAD = 4


@functools.lru_cache(maxsize=None)
def _make_gather(N, S):
    n_per_w = N // NW
    ngroup = n_per_w // NBUF
    assert n_per_w % NBUF == 0 and ngroup >= 3
    mesh = plsc.VectorSubcoreMesh(core_axis_name="c", subcore_axis_name="s")

    @functools.partial(
        pl.kernel,
        mesh=mesh,
        out_type=jax.ShapeDtypeStruct((N, S, EMB), jnp.float32),
        scratch_types=[
            pltpu.VMEM((n_per_w, S), jnp.int32),
            pltpu.VMEM((NBUF, S, EMB), jnp.float32),
        ]
        + [pltpu.SemaphoreType.DMA] * (2 * NBUF),
        compiler_params=pltpu.CompilerParams(use_tc_tiling_on_sc=True),
    )
    def gather_kernel(idx_hbm, table_hbm, out_hbm, idx_v, rows_v, *sems):
        gsems, ssems = sems[:NBUF], sems[NBUF:]
        wid = lax.axis_index("s") * NC + lax.axis_index("c")
        pltpu.sync_copy(idx_hbm.at[wid], idx_v)
        base = wid * n_per_w

        def issue_gather(i, j):
            pltpu.async_copy(table_hbm.at[idx_v.at[i]], rows_v.at[j], gsems[j])

        def wait_gather(j):
            pltpu.make_async_copy(
                table_hbm.at[idx_v.at[0]], rows_v.at[j], gsems[j]
            ).wait()

        def issue_store(i, j):
            pltpu.async_copy(rows_v.at[j], out_hbm.at[base + i], ssems[j])

        def wait_store(i, j):
            pltpu.make_async_copy(
                rows_v.at[j], out_hbm.at[base + i], ssems[j]
            ).wait()

        def step(i, b, do_wait, do_issue):
            s = (b + AHEAD) % NBUF
            if do_wait:
                wait_store(i + AHEAD - NBUF, s)
            if do_issue:
                issue_gather(i + AHEAD, s)
            wait_gather(b)
            issue_store(i, b)

        for j in range(AHEAD):
            issue_gather(j, j)

        for b in range(NBUF):
            step(b, b, do_wait=b + AHEAD >= NBUF, do_issue=True)

        def group_body(g, carry):
            for b in range(NBUF):
                step(g * NBUF + b, b, do_wait=True, do_issue=True)
            return carry

        lax.fori_loop(1, ngroup - 1, group_body, 0)

        for b in range(NBUF):
            i = (ngroup - 1) * NBUF + b
            step(i, b, do_wait=True, do_issue=i + AHEAD < n_per_w)
        for i in range(n_per_w - NBUF + AHEAD, n_per_w):
            wait_store(i, i % NBUF)

    return gather_kernel


def kernel(input_tensor, table):
    n, s = input_tensor.shape
    idx = input_tensor.reshape(NW, n // NW, s)
    return _make_gather(n, s)(idx, table)
